# baseline (speedup 1.0000x reference)
import functools

import jax
import jax.numpy as jnp
from jax import lax
from jax.experimental import pallas as pl
from jax.experimental.pallas import tpu as pltpu
from jax.experimental.pallas import tpu_sc as plsc

NUM_CORES = 2
NUM_SUBCORES = 16
BATCH = 128
ROUNDS = 2
PAD = 256


def _make_scatter_kernel(num_slots: int, num_indices: int):
  assert num_indices % (NUM_SUBCORES * BATCH) == 0
  assert num_slots % (NUM_CORES * ROUNDS) == 0
  rows_per_tile = num_indices // (NUM_SUBCORES * BATCH)
  quarter = num_slots // (NUM_CORES * ROUNDS)
  chunk = (quarter // NUM_SUBCORES) // 8 * 8
  tail = quarter - chunk * NUM_SUBCORES
  n_sub = 3
  assert chunk % (n_sub * 8) == 0
  sub = chunk // n_sub

  mesh = plsc.VectorSubcoreMesh(
      core_axis_name="c", subcore_axis_name="s", num_cores=NUM_CORES)

  @functools.partial(
      pl.kernel,
      out_type=jax.ShapeDtypeStruct((num_slots,), jnp.float32),
      mesh=mesh,
      scratch_types=[
          pltpu.VMEM((rows_per_tile, BATCH), jnp.int32),
          pltpu.VMEM((BATCH,), jnp.float32),
          pltpu.VMEM((16,), jnp.float32),
          pltpu.VMEM((sub,), jnp.float32),
          pltpu.VMEM((sub,), jnp.float32),
          pltpu.VMEM_SHARED((quarter + PAD,), jnp.float32),
          pltpu.SemaphoreType.DMA,
          pltpu.SemaphoreType.DMA,
          pltpu.SemaphoreType.DMA,
          pltpu.SemaphoreType.DMA,
      ],
  )
  def scatter_kernel(steps_hbm, idx_hbm, gs_hbm, out_hbm,
                     idx_v, val_v, gs_v, buf0, buf1, table_sh,
                     idx_sem, buf_sem0, buf_sem1, sem):
    cid = lax.axis_index("c")
    sid = lax.axis_index("s")

    idx_cp = pltpu.async_copy(idx_hbm.at[sid], idx_v, idx_sem)

    pltpu.sync_copy(gs_hbm, gs_v)
    gs_vec = gs_v[...]
    for i in range(BATCH // 16):
      val_v[pl.ds(i * 16, 16)] = gs_vec

    off = sid * chunk
    bufs = (buf0, buf1)
    sems = (buf_sem0, buf_sem1)
    quarter_vec = jnp.full((16,), quarter, jnp.int32)
    quarter_uvec = jnp.full((16,), quarter, jnp.uint32)
    zero_vec = jnp.full((16,), 0, jnp.int32)
    mask_vec = jnp.full((16,), PAD - 1, jnp.int32)

    for r in range(ROUNDS):
      base = (cid + NUM_CORES * r) * quarter
      base_vec = zero_vec + base

      loads = [None, None]
      loads[0] = pltpu.async_copy(
          steps_hbm.at[pl.ds(base + off, sub)], buf0, buf_sem0)
      for k in range(n_sub):
        if k + 1 < n_sub:
          loads[(k + 1) % 2] = pltpu.async_copy(
              steps_hbm.at[pl.ds(base + off + (k + 1) * sub, sub)],
              bufs[(k + 1) % 2], sems[(k + 1) % 2])
        loads[k % 2].wait()
        pltpu.sync_copy(bufs[k % 2], table_sh.at[pl.ds(off + k * sub, sub)])
      if tail:
        @pl.when(sid == NUM_SUBCORES - 1)
        def _():
          pltpu.sync_copy(
              steps_hbm.at[pl.ds(base + chunk * NUM_SUBCORES, tail)],
              buf0.at[pl.ds(0, tail)])
          pltpu.sync_copy(buf0.at[pl.ds(0, tail)],
                          table_sh.at[pl.ds(chunk * NUM_SUBCORES, tail)])

      idx_cp.wait()

      def clamp_body(g, _):
        for j in range(BATCH // 16):
          iv = idx_v[g, pl.ds(j * 16, 16)] - base_vec
          ok = plsc.bitcast(iv, jnp.uint32) < quarter_uvec
          dummy = quarter_vec + (iv & mask_vec)
          idx_v[g, pl.ds(j * 16, 16)] = jnp.where(ok, iv, dummy)
        return 0

      lax.fori_loop(0, rows_per_tile, clamp_body, 0)

      plsc.subcore_barrier()

      def issue_body(g, _):
        pltpu.async_copy(val_v, table_sh.at[idx_v.at[g]], sem)
        return 0

      lax.fori_loop(0, rows_per_tile, issue_body, 0)

      def drain_body(g, _):
        pltpu.make_async_copy(val_v, table_sh.at[idx_v.at[0]], sem).wait()
        return 0

      lax.fori_loop(0, rows_per_tile, drain_body, 0)

      plsc.subcore_barrier()

      if r + 1 < ROUNDS:
        idx_cp = pltpu.async_copy(idx_hbm.at[sid], idx_v, idx_sem)

      stores = [None, None]
      for k in range(n_sub):
        if stores[k % 2] is not None:
          stores[k % 2].wait()
        pltpu.sync_copy(table_sh.at[pl.ds(off + k * sub, sub)], bufs[k % 2])
        stores[k % 2] = pltpu.async_copy(
            bufs[k % 2], out_hbm.at[pl.ds(base + off + k * sub, sub)],
            sems[k % 2])
      for d in stores:
        d.wait()
      if tail:
        @pl.when(sid == NUM_SUBCORES - 1)
        def _():
          pltpu.sync_copy(table_sh.at[pl.ds(chunk * NUM_SUBCORES, tail)],
                          buf0.at[pl.ds(0, tail)])
          pltpu.sync_copy(
              buf0.at[pl.ds(0, tail)],
              out_hbm.at[pl.ds(base + chunk * NUM_SUBCORES, tail)])

  return scatter_kernel


def kernel(steps, index, global_step):
  num_slots = steps.shape[0]
  num_indices = index.shape[0]
  rows_per_tile = num_indices // (NUM_SUBCORES * BATCH)
  idx3 = jnp.reshape(index, (NUM_SUBCORES, rows_per_tile, BATCH))
  gs = jnp.full((16,), global_step, dtype=jnp.float32)
  f = _make_scatter_kernel(num_slots, num_indices)
  return f(steps, idx3, gs)


# speedup vs baseline: 1.0023x; 1.0023x over previous
import functools

import jax
import jax.numpy as jnp
from jax import lax
from jax.experimental import pallas as pl
from jax.experimental.pallas import tpu as pltpu
from jax.experimental.pallas import tpu_sc as plsc

NUM_CORES = 2
NUM_SUBCORES = 16
BATCH = 128
ROUNDS = 2
PAD = 2048


def _make_scatter_kernel(num_slots: int, num_indices: int):
  assert num_indices % (NUM_SUBCORES * BATCH) == 0
  assert num_slots % (NUM_CORES * ROUNDS) == 0
  rows_per_tile = num_indices // (NUM_SUBCORES * BATCH)
  quarter = num_slots // (NUM_CORES * ROUNDS)
  chunk = (quarter // NUM_SUBCORES) // 8 * 8
  tail = quarter - chunk * NUM_SUBCORES
  n_sub = 3
  assert chunk % (n_sub * 8) == 0
  sub = chunk // n_sub

  mesh = plsc.VectorSubcoreMesh(
      core_axis_name="c", subcore_axis_name="s", num_cores=NUM_CORES)

  @functools.partial(
      pl.kernel,
      out_type=jax.ShapeDtypeStruct((num_slots,), jnp.float32),
      mesh=mesh,
      scratch_types=[
          pltpu.VMEM((rows_per_tile, BATCH), jnp.int32),
          pltpu.VMEM((BATCH,), jnp.float32),
          pltpu.VMEM((16,), jnp.float32),
          pltpu.VMEM((sub,), jnp.float32),
          pltpu.VMEM((sub,), jnp.float32),
          pltpu.VMEM_SHARED((quarter + PAD,), jnp.float32),
          pltpu.SemaphoreType.DMA,
          pltpu.SemaphoreType.DMA,
          pltpu.SemaphoreType.DMA,
          pltpu.SemaphoreType.DMA,
      ],
  )
  def scatter_kernel(steps_hbm, idx_hbm, gs_hbm, out_hbm,
                     idx_v, val_v, gs_v, buf0, buf1, table_sh,
                     idx_sem, buf_sem0, buf_sem1, sem):
    cid = lax.axis_index("c")
    sid = lax.axis_index("s")

    idx_cp = pltpu.async_copy(idx_hbm.at[sid], idx_v, idx_sem)

    pltpu.sync_copy(gs_hbm, gs_v)
    gs_vec = gs_v[...]
    for i in range(BATCH // 16):
      val_v[pl.ds(i * 16, 16)] = gs_vec

    off = sid * chunk
    bufs = (buf0, buf1)
    sems = (buf_sem0, buf_sem1)
    quarter_vec = jnp.full((16,), quarter, jnp.int32)
    quarter_uvec = jnp.full((16,), quarter, jnp.uint32)
    zero_vec = jnp.full((16,), 0, jnp.int32)
    mask_vec = jnp.full((16,), PAD - 1, jnp.int32)

    for r in range(ROUNDS):
      base = (cid + NUM_CORES * r) * quarter
      base_vec = zero_vec + base

      loads = [None, None]
      loads[0] = pltpu.async_copy(
          steps_hbm.at[pl.ds(base + off, sub)], buf0, buf_sem0)
      for k in range(n_sub):
        if k + 1 < n_sub:
          loads[(k + 1) % 2] = pltpu.async_copy(
              steps_hbm.at[pl.ds(base + off + (k + 1) * sub, sub)],
              bufs[(k + 1) % 2], sems[(k + 1) % 2])
        loads[k % 2].wait()
        pltpu.sync_copy(bufs[k % 2], table_sh.at[pl.ds(off + k * sub, sub)])
      if tail:
        @pl.when(sid == NUM_SUBCORES - 1)
        def _():
          pltpu.sync_copy(
              steps_hbm.at[pl.ds(base + chunk * NUM_SUBCORES, tail)],
              buf0.at[pl.ds(0, tail)])
          pltpu.sync_copy(buf0.at[pl.ds(0, tail)],
                          table_sh.at[pl.ds(chunk * NUM_SUBCORES, tail)])

      idx_cp.wait()

      def clamp_body(h, _):
        for rr in range(2):
          g = h * 2 + rr
          for j in range(BATCH // 16):
            iv = idx_v[g, pl.ds(j * 16, 16)] - base_vec
            ok = plsc.bitcast(iv, jnp.uint32) < quarter_uvec
            dummy = quarter_vec + (iv & mask_vec)
            idx_v[g, pl.ds(j * 16, 16)] = jnp.where(ok, iv, dummy)
        return 0

      lax.fori_loop(0, rows_per_tile // 2, clamp_body, 0)

      plsc.subcore_barrier()

      def issue_body(g, _):
        pltpu.async_copy(val_v, table_sh.at[idx_v.at[g]], sem)
        return 0

      lax.fori_loop(0, rows_per_tile, issue_body, 0)

      def drain_body(g, _):
        pltpu.make_async_copy(val_v, table_sh.at[idx_v.at[0]], sem).wait()
        return 0

      lax.fori_loop(0, rows_per_tile, drain_body, 0)

      plsc.subcore_barrier()

      if r + 1 < ROUNDS:
        idx_cp = pltpu.async_copy(idx_hbm.at[sid], idx_v, idx_sem)

      stores = [None, None]
      for k in range(n_sub):
        if stores[k % 2] is not None:
          stores[k % 2].wait()
        pltpu.sync_copy(table_sh.at[pl.ds(off + k * sub, sub)], bufs[k % 2])
        stores[k % 2] = pltpu.async_copy(
            bufs[k % 2], out_hbm.at[pl.ds(base + off + k * sub, sub)],
            sems[k % 2])
      for d in stores:
        d.wait()
      if tail:
        @pl.when(sid == NUM_SUBCORES - 1)
        def _():
          pltpu.sync_copy(table_sh.at[pl.ds(chunk * NUM_SUBCORES, tail)],
                          buf0.at[pl.ds(0, tail)])
          pltpu.sync_copy(
              buf0.at[pl.ds(0, tail)],
              out_hbm.at[pl.ds(base + chunk * NUM_SUBCORES, tail)])

  return scatter_kernel


def kernel(steps, index, global_step):
  num_slots = steps.shape[0]
  num_indices = index.shape[0]
  rows_per_tile = num_indices // (NUM_SUBCORES * BATCH)
  idx3 = jnp.reshape(index, (NUM_SUBCORES, rows_per_tile, BATCH))
  gs = jnp.full((16,), global_step, dtype=jnp.float32)
  f = _make_scatter_kernel(num_slots, num_indices)
  return f(steps, idx3, gs)
